# Initial kernel scaffold; baseline (speedup 1.0000x reference)
import jax
import jax.numpy as jnp
from jax.experimental import pallas as pl


def kernel(x, conv_w, conv_b):
    raise NotImplementedError("write your pallas kernel here")


# speedup vs baseline: 24.4560x; 24.4560x over previous
import functools

import jax
import jax.numpy as jnp
from jax.experimental import pallas as pl

_K = 9
_NEG_INF = float("-inf")


def _dygraph_kernel(xt_ref, w_ref, b_ref, out_ref):
    xt = xt_ref[0]
    n, c = xt.shape
    cout = w_ref.shape[0]
    half = cout // 2
    cg = c // 2

    nrm = jnp.sqrt(jnp.sum(xt * xt, axis=1, keepdims=True))
    xn = xt / jnp.maximum(nrm, 1e-12)
    x_sq = jnp.sum(xn * xn, axis=1, keepdims=True)
    inner = jax.lax.dot_general(
        xn, xn, (((1,), (1,)), ((), ())),
        preferred_element_type=jnp.float32)
    neg_dist = 2.0 * inner - x_sq - jnp.transpose(x_sq)

    w0 = w_ref[0:cg, :]
    w1 = w_ref[cg:2 * cg, :]
    w2 = w_ref[2 * cg:3 * cg, :]
    w3 = w_ref[3 * cg:4 * cg, :]
    xa = xt[:, :cg]
    xb = xt[:, cg:]

    def mm(a, b):
        return jax.lax.dot_general(
            a, b, (((1,), (1,)), ((), ())),
            preferred_element_type=jnp.float32)

    y_u = jnp.concatenate([mm(xa, w0), mm(xb, w1)], axis=1)
    y_v = jnp.concatenate([mm(xa, w2), mm(xb, w3)], axis=1)

    col = jax.lax.broadcasted_iota(jnp.int32, (n, n), 1)

    def body(_, carry):
        nd, agg = carry
        m = jnp.max(nd, axis=1, keepdims=True)
        idx = jnp.min(jnp.where(nd == m, col, n), axis=1,
                      keepdims=True)
        onehot = (col == idx)
        gath = jax.lax.dot_general(
            onehot.astype(jnp.float32), y_v,
            (((1,), (0,)), ((), ())),
            preferred_element_type=jnp.float32)
        agg = jnp.maximum(agg, gath)
        nd = jnp.where(onehot, _NEG_INF, nd)
        return nd, agg

    agg0 = jnp.full((n, half), _NEG_INF, dtype=jnp.float32)
    _, agg = jax.lax.fori_loop(0, _K, body, (neg_dist, agg0))

    b_top = b_ref[0, :half][None, :]
    b_bot = b_ref[0, half:][None, :]
    out_top = jnp.maximum(y_u + b_top, 0.0)
    out_bot = jnp.maximum(agg - y_v + b_bot, 0.0)
    out_ref[0] = jnp.concatenate([out_top, out_bot], axis=1)


@jax.jit
def kernel(x, conv_w, conv_b):
    B, C, H, W = x.shape
    N = H * W
    Cout = conv_w.shape[0]
    xt = jnp.transpose(x.reshape(B, C, N), (0, 2, 1))

    out = pl.pallas_call(
        _dygraph_kernel,
        grid=(B,),
        in_specs=[
            pl.BlockSpec((1, N, C), lambda b: (b, 0, 0)),
            pl.BlockSpec((Cout, conv_w.shape[1]), lambda b: (0, 0)),
            pl.BlockSpec((1, Cout), lambda b: (0, 0)),
        ],
        out_specs=pl.BlockSpec((1, N, Cout), lambda b: (b, 0, 0)),
        out_shape=jax.ShapeDtypeStruct((B, N, Cout), jnp.float32),
    )(xt, conv_w, conv_b.reshape(1, Cout))

    return jnp.transpose(out, (0, 2, 1)).reshape(B, Cout, H, W)
